# baseline (speedup 1.0000x reference)
import functools

import jax
import jax.numpy as jnp
from jax import lax
from jax.experimental import pallas as pl
from jax.experimental.pallas import tpu as pltpu
from jax.experimental.pallas import tpu_sc as plsc

Q = 4096
KEYS = 65536
D = 768
TOPK = 4

QB = 512
KB = 2048

SC_CORES = 2
SC_SUBCORES = 16
SC_WORKERS = SC_CORES * SC_SUBCORES
GATHER_ROWS = Q * TOPK
ROWS_PER_WORKER = GATHER_ROWS // SC_WORKERS
CHUNK = 64
NCHUNKS = ROWS_PER_WORKER // CHUNK


def _topk_body(z_ref, tgt_ref, cv_ref, cp_ref, tn_s, cvs, cps, colf_s):
    kb = pl.program_id(0)
    qb = pl.program_id(1)

    @pl.when(jnp.logical_and(kb == 0, qb == 0))
    def _():
        colf_s[...] = lax.broadcasted_iota(
            jnp.int32, (QB, KB), 1).astype(jnp.float32)

    @pl.when(qb == 0)
    def _():
        t = tgt_ref[...]
        tn_s[...] = t * lax.rsqrt(jnp.sum(t * t, axis=-1, keepdims=True) + 1e-8)

    z = z_ref[...]
    zn = z * lax.rsqrt(jnp.sum(z * z, axis=-1, keepdims=True) + 1e-8)
    sim = lax.dot_general(zn, tn_s[...], (((1,), (1,)), ((), ())),
                          preferred_element_type=jnp.float32)

    colf = colf_s[...]
    basef = (pl.program_id(0) * KB).astype(jnp.float32)
    tvs, tis = [], []
    s = sim
    for _ in range(TOPK):
        m = jnp.max(s, axis=1, keepdims=True)
        pick = jnp.min(jnp.where(s == m, colf, float(KEYS)), axis=1, keepdims=True)
        tvs.append(m)
        tis.append(pick + basef)
        s = jnp.where(colf == pick, -jnp.inf, s)

    nc = pl.num_programs(0) * TOPK
    lane = lax.broadcasted_iota(jnp.int32, (QB, nc), 1)
    vacc = cvs[pl.ds(qb * QB, QB), :]
    iacc = cps[pl.ds(qb * QB, QB), :]
    for r in range(TOPK):
        vacc = jnp.where(lane == kb * TOPK + r, tvs[r], vacc)
        iacc = jnp.where(lane == kb * TOPK + r, tis[r], iacc)
    cvs[pl.ds(qb * QB, QB), :] = vacc
    cps[pl.ds(qb * QB, QB), :] = iacc

    @pl.when(kb == pl.num_programs(0) - 1)
    def _():
        cv_ref[...] = vacc
        cp_ref[...] = iacc


def _topk_call(z, tgt):
    nkb = KEYS // KB
    return pl.pallas_call(
        _topk_body,
        grid=(nkb, Q // QB),
        in_specs=[
            pl.BlockSpec((QB, D), lambda kb, qb: (qb, 0)),
            pl.BlockSpec((KB, D), lambda kb, qb: (kb, 0)),
        ],
        out_specs=[
            pl.BlockSpec((QB, nkb * TOPK), lambda kb, qb: (qb, 0)),
            pl.BlockSpec((QB, nkb * TOPK), lambda kb, qb: (qb, 0)),
        ],
        out_shape=[
            jax.ShapeDtypeStruct((Q, nkb * TOPK), jnp.float32),
            jax.ShapeDtypeStruct((Q, nkb * TOPK), jnp.float32),
        ],
        scratch_shapes=[
            pltpu.VMEM((KB, D), jnp.float32),
            pltpu.VMEM((Q, nkb * TOPK), jnp.float32),
            pltpu.VMEM((Q, nkb * TOPK), jnp.float32),
            pltpu.VMEM((QB, KB), jnp.float32),
        ],
    )(z, tgt)


def _merge_body(cv_ref, cp_ref, idx_ref):
    v = cv_ref[...]
    gi = cp_ref[...]
    outs = []
    for _ in range(TOPK):
        m = jnp.max(v, axis=1, keepdims=True)
        pickid = jnp.min(jnp.where(v == m, gi, float(2 * KEYS)), axis=1,
                         keepdims=True)
        outs.append(pickid)
        v = jnp.where(gi == pickid, -jnp.inf, v)
    idx_ref[...] = jnp.concatenate(outs, axis=1).astype(jnp.int32)


def _merge_call(cv2, cp2):
    nc = (KEYS // KB) * TOPK
    return pl.pallas_call(
        _merge_body,
        grid=(Q // QB,),
        in_specs=[
            pl.BlockSpec((QB, nc), lambda q: (q, 0)),
            pl.BlockSpec((QB, nc), lambda q: (q, 0)),
        ],
        out_specs=pl.BlockSpec((QB, TOPK), lambda q: (q, 0)),
        out_shape=jax.ShapeDtypeStruct((Q, TOPK), jnp.int32),
    )(cv2, cp2)


def _gather_body(tgt_hbm, idx_hbm, out_hbm, idx_v, rows0, rows1, sem0, sem1):
    wid = lax.axis_index("s") * SC_CORES + lax.axis_index("c")
    base = wid * ROWS_PER_WORKER
    pltpu.sync_copy(idx_hbm.at[pl.ds(base, ROWS_PER_WORKER)], idx_v)
    bufs = (rows0, rows1)
    sems = (sem0, sem1)
    prev = pltpu.async_copy(tgt_hbm.at[idx_v.at[pl.ds(0, CHUNK)]], rows0, sem0)
    for c in range(1, NCHUNKS + 1):
        if c < NCHUNKS:
            cur = pltpu.async_copy(
                tgt_hbm.at[idx_v.at[pl.ds(c * CHUNK, CHUNK)]],
                bufs[c % 2], sems[c % 2])
        prev.wait()
        pltpu.sync_copy(bufs[(c - 1) % 2],
                        out_hbm.at[pl.ds(base + (c - 1) * CHUNK, CHUNK)])
        if c < NCHUNKS:
            prev = cur


def _gather_call(tgt, idx_flat):
    fn = functools.partial(
        pl.kernel,
        mesh=plsc.VectorSubcoreMesh(core_axis_name="c", subcore_axis_name="s"),
        out_type=jax.ShapeDtypeStruct((GATHER_ROWS, D), jnp.float32),
        scratch_types=[
            pltpu.VMEM((ROWS_PER_WORKER,), jnp.int32),
            pltpu.VMEM((CHUNK, D), jnp.float32),
            pltpu.VMEM((CHUNK, D), jnp.float32),
            pltpu.SemaphoreType.DMA,
            pltpu.SemaphoreType.DMA,
        ],
    )(_gather_body)
    return fn(tgt, idx_flat)


def _mean_body(g_ref, o_ref):
    g = g_ref[...]
    o_ref[...] = (g[:, :D] + g[:, D:2 * D] + g[:, 2 * D:3 * D] + g[:, 3 * D:]) * 0.25


def _mean_call(g2):
    return pl.pallas_call(
        _mean_body,
        grid=(Q // QB,),
        in_specs=[pl.BlockSpec((QB, TOPK * D), lambda i: (i, 0))],
        out_specs=pl.BlockSpec((QB, D), lambda i: (i, 0)),
        out_shape=jax.ShapeDtypeStruct((Q, D), jnp.float32),
    )(g2)


def kernel(z, tgt, k):
    del k
    cv, cp = _topk_call(z, tgt)
    idx = _merge_call(cv, cp)
    g = _gather_call(tgt, idx.reshape(GATHER_ROWS))
    return _mean_call(g.reshape(Q, TOPK * D))


# speedup vs baseline: 1.0228x; 1.0228x over previous
import functools

import jax
import jax.numpy as jnp
from jax import lax
from jax.experimental import pallas as pl
from jax.experimental.pallas import tpu as pltpu
from jax.experimental.pallas import tpu_sc as plsc

Q = 4096
KEYS = 65536
D = 768
TOPK = 4

QB = 512
KB = 2048

SC_CORES = 2
SC_SUBCORES = 16
SC_WORKERS = SC_CORES * SC_SUBCORES
GATHER_ROWS = Q * TOPK
ROWS_PER_WORKER = GATHER_ROWS // SC_WORKERS
CHUNK = 64
NCHUNKS = ROWS_PER_WORKER // CHUNK


def _topk_body(z_ref, tgt_ref, cv_ref, cp_ref, tn_s, cvs, cps):
    kb = pl.program_id(0)
    qb = pl.program_id(1)

    @pl.when(qb == 0)
    def _():
        t = tgt_ref[...]
        tn_s[...] = t * lax.rsqrt(jnp.sum(t * t, axis=-1, keepdims=True) + 1e-8)

    z = z_ref[...]
    zn = z * lax.rsqrt(jnp.sum(z * z, axis=-1, keepdims=True) + 1e-8)
    sim = lax.dot_general(zn, tn_s[...], (((1,), (1,)), ((), ())),
                          preferred_element_type=jnp.float32)

    colf = lax.broadcasted_iota(jnp.int32, (QB, KB), 1).astype(jnp.float32)
    basef = (pl.program_id(0) * KB).astype(jnp.float32)
    tvs, tis = [], []
    s = sim
    for _ in range(TOPK):
        m = jnp.max(s, axis=1, keepdims=True)
        pick = jnp.min(jnp.where(s == m, colf, float(KEYS)), axis=1, keepdims=True)
        tvs.append(m)
        tis.append(pick + basef)
        s = jnp.where(colf == pick, -jnp.inf, s)

    nc = pl.num_programs(0) * TOPK
    lane = lax.broadcasted_iota(jnp.int32, (QB, nc), 1)
    vacc = cvs[pl.ds(qb * QB, QB), :]
    iacc = cps[pl.ds(qb * QB, QB), :]
    for r in range(TOPK):
        vacc = jnp.where(lane == kb * TOPK + r, tvs[r], vacc)
        iacc = jnp.where(lane == kb * TOPK + r, tis[r], iacc)
    cvs[pl.ds(qb * QB, QB), :] = vacc
    cps[pl.ds(qb * QB, QB), :] = iacc

    @pl.when(kb == pl.num_programs(0) - 1)
    def _():
        cv_ref[...] = vacc
        cp_ref[...] = iacc


def _topk_call(z, tgt):
    nkb = KEYS // KB
    return pl.pallas_call(
        _topk_body,
        grid=(nkb, Q // QB),
        in_specs=[
            pl.BlockSpec((QB, D), lambda kb, qb: (qb, 0)),
            pl.BlockSpec((KB, D), lambda kb, qb: (kb, 0)),
        ],
        out_specs=[
            pl.BlockSpec((QB, nkb * TOPK), lambda kb, qb: (qb, 0)),
            pl.BlockSpec((QB, nkb * TOPK), lambda kb, qb: (qb, 0)),
        ],
        out_shape=[
            jax.ShapeDtypeStruct((Q, nkb * TOPK), jnp.float32),
            jax.ShapeDtypeStruct((Q, nkb * TOPK), jnp.float32),
        ],
        scratch_shapes=[
            pltpu.VMEM((KB, D), jnp.float32),
            pltpu.VMEM((Q, nkb * TOPK), jnp.float32),
            pltpu.VMEM((Q, nkb * TOPK), jnp.float32),
        ],
    )(z, tgt)


def _merge_body(cv_ref, cp_ref, idx_ref):
    v = cv_ref[...]
    gi = cp_ref[...]
    outs = []
    for _ in range(TOPK):
        m = jnp.max(v, axis=1, keepdims=True)
        pickid = jnp.min(jnp.where(v == m, gi, float(2 * KEYS)), axis=1,
                         keepdims=True)
        outs.append(pickid)
        v = jnp.where(gi == pickid, -jnp.inf, v)
    idx_ref[...] = jnp.concatenate(outs, axis=1).astype(jnp.int32)


def _merge_call(cv2, cp2):
    nc = (KEYS // KB) * TOPK
    return pl.pallas_call(
        _merge_body,
        grid=(Q // QB,),
        in_specs=[
            pl.BlockSpec((QB, nc), lambda q: (q, 0)),
            pl.BlockSpec((QB, nc), lambda q: (q, 0)),
        ],
        out_specs=pl.BlockSpec((QB, TOPK), lambda q: (q, 0)),
        out_shape=jax.ShapeDtypeStruct((Q, TOPK), jnp.int32),
    )(cv2, cp2)


def _gather_body(tgt_hbm, idx_hbm, out_hbm, idx_v, rows0, rows1, sem0, sem1):
    wid = lax.axis_index("s") * SC_CORES + lax.axis_index("c")
    base = wid * ROWS_PER_WORKER
    pltpu.sync_copy(idx_hbm.at[pl.ds(base, ROWS_PER_WORKER)], idx_v)
    bufs = (rows0, rows1)
    sems = (sem0, sem1)
    prev = pltpu.async_copy(tgt_hbm.at[idx_v.at[pl.ds(0, CHUNK)]], rows0, sem0)
    for c in range(1, NCHUNKS + 1):
        if c < NCHUNKS:
            cur = pltpu.async_copy(
                tgt_hbm.at[idx_v.at[pl.ds(c * CHUNK, CHUNK)]],
                bufs[c % 2], sems[c % 2])
        prev.wait()
        pltpu.sync_copy(bufs[(c - 1) % 2],
                        out_hbm.at[pl.ds(base + (c - 1) * CHUNK, CHUNK)])
        if c < NCHUNKS:
            prev = cur


def _gather_call(tgt, idx_flat):
    fn = functools.partial(
        pl.kernel,
        mesh=plsc.VectorSubcoreMesh(core_axis_name="c", subcore_axis_name="s"),
        out_type=jax.ShapeDtypeStruct((GATHER_ROWS, D), jnp.float32),
        scratch_types=[
            pltpu.VMEM((ROWS_PER_WORKER,), jnp.int32),
            pltpu.VMEM((CHUNK, D), jnp.float32),
            pltpu.VMEM((CHUNK, D), jnp.float32),
            pltpu.SemaphoreType.DMA,
            pltpu.SemaphoreType.DMA,
        ],
    )(_gather_body)
    return fn(tgt, idx_flat)


def _mean_body(g_ref, o_ref):
    g = g_ref[...]
    o_ref[...] = (g[:, :D] + g[:, D:2 * D] + g[:, 2 * D:3 * D] + g[:, 3 * D:]) * 0.25


def _mean_call(g2):
    return pl.pallas_call(
        _mean_body,
        grid=(Q // QB,),
        in_specs=[pl.BlockSpec((QB, TOPK * D), lambda i: (i, 0))],
        out_specs=pl.BlockSpec((QB, D), lambda i: (i, 0)),
        out_shape=jax.ShapeDtypeStruct((Q, D), jnp.float32),
    )(g2)


def kernel(z, tgt, k):
    del k
    cv, cp = _topk_call(z, tgt)
    idx = _merge_call(cv, cp)
    g = _gather_call(tgt, idx.reshape(GATHER_ROWS))
    return _mean_call(g.reshape(Q, TOPK * D))
